# baseline (speedup 1.0000x reference)
import functools

import jax
import jax.numpy as jnp
from jax import lax
from jax.experimental import pallas as pl
from jax.experimental.pallas import tpu as pltpu
from jax.experimental.pallas import tpu_sc as plsc

_N = 10000
_E = 320000
_HID = 128
_NUM_LAYERS = 3
_MIN_DISP_SCALE = 0.001
_CLAMP_LO, _CLAMP_HI = 0.0, 30.0
_YIELD_STRESS = 2.5e8

_NC = 2
_NS = 16
_NW = _NC * _NS
_EPW = _E // _NW
_CHUNK = 80
_BLK_CHUNKS = 1
_BLKE = _CHUNK * _BLK_CHUNKS
_NBLK = _EPW // _BLKE
_ROWS_PER_TILE = 624

_NB = 2000
_EB = 2000


def _ln(h, g, b):
    m = jnp.mean(h, axis=-1, keepdims=True)
    v = jnp.mean((h - m) ** 2, axis=-1, keepdims=True)
    return (h - m) / jnp.sqrt(v + 1e-5) * g + b


def _dot(a, b):
    return jnp.dot(a, b, preferred_element_type=jnp.float32)


def _pack_words(x):
    u = jax.lax.bitcast_convert_type(x, jnp.uint32)
    r = (u + 0x7FFF + ((u >> 16) & 1)) >> 16
    lo = jnp.concatenate([r[:, g * 32:g * 32 + 16] for g in range(4)], axis=1)
    hi = jnp.concatenate([r[:, g * 32 + 16:g * 32 + 32] for g in range(4)],
                         axis=1)
    return jax.lax.bitcast_convert_type((hi << 16) | lo, jnp.float32)


def _node_enc_body(x_ref, w1, b1, w2, b2, g, b, o_ref):
    h = jnp.maximum(_dot(x_ref[...], w1[...]) + b1[...], 0.0)
    h = _dot(h, w2[...]) + b2[...]
    o_ref[...] = _ln(h, g[...], b[...])


def _node_encoder(x, p, ln):
    spec128 = pl.BlockSpec((_HID, _HID), lambda i: (0, 0))
    row = pl.BlockSpec((1, _HID), lambda i: (0, 0))
    return pl.pallas_call(
        _node_enc_body,
        grid=(_N // _NB,),
        in_specs=[
            pl.BlockSpec((_NB, x.shape[1]), lambda i: (i, 0)),
            pl.BlockSpec(p[0]["w"].shape, lambda i: (0, 0)),
            row, spec128, row, row, row,
        ],
        out_specs=pl.BlockSpec((_NB, _HID), lambda i: (i, 0)),
        out_shape=jax.ShapeDtypeStruct((_N, _HID), jnp.float32),
    )(x, p[0]["w"], p[0]["b"].reshape(1, -1), p[1]["w"],
      p[1]["b"].reshape(1, -1), ln["g"].reshape(1, -1), ln["b"].reshape(1, -1))


def _edge_enc_body(a_ref, w1, b1, w2, b2, g, b, lw, lb, o1, o2, o3):
    e = jnp.maximum(_dot(a_ref[...], w1[...]) + b1[...], 0.0)
    e = _dot(e, w2[...]) + b2[...]
    e = _ln(e, g[...], b[...])
    o1[...] = _dot(e, lw[0]) + lb[0]
    o2[...] = _dot(e, lw[1]) + lb[1]
    o3[...] = _dot(e, lw[2]) + lb[2]


def _edge_encoder(edge_attr, p, ln, convs):
    lw = jnp.stack([c["edge_lin"]["w"] for c in convs])
    lb = jnp.stack([c["edge_lin"]["b"].reshape(1, -1) for c in convs])
    spec128 = pl.BlockSpec((_HID, _HID), lambda i: (0, 0))
    row = pl.BlockSpec((1, _HID), lambda i: (0, 0))
    out_spec = pl.BlockSpec((_EB, _HID), lambda i: (i, 0))
    out_sh = jax.ShapeDtypeStruct((_E, _HID), jnp.float32)
    return pl.pallas_call(
        _edge_enc_body,
        grid=(_E // _EB,),
        in_specs=[
            pl.BlockSpec((_EB, edge_attr.shape[1]), lambda i: (i, 0)),
            pl.BlockSpec(p[0]["w"].shape, lambda i: (0, 0)),
            row, spec128, row, row, row,
            pl.BlockSpec((_NUM_LAYERS, _HID, _HID), lambda i: (0, 0, 0)),
            pl.BlockSpec((_NUM_LAYERS, 1, _HID), lambda i: (0, 0, 0)),
        ],
        out_specs=[out_spec, out_spec, out_spec],
        out_shape=[out_sh, out_sh, out_sh],
    )(edge_attr, p[0]["w"], p[0]["b"].reshape(1, -1), p[1]["w"],
      p[1]["b"].reshape(1, -1), ln["g"].reshape(1, -1), ln["b"].reshape(1, -1),
      lw, lb)


def _node_update_body(h_ref, agg_ref, w1, b1, w2, b2, g, b, o_ref):
    h = h_ref[...]
    z = h + agg_ref[0] + agg_ref[1]
    t = jnp.maximum(_dot(z, w1[...]) + b1[...], 0.0)
    t = _dot(t, w2[...]) + b2[...]
    t = jnp.maximum(t, 0.0)
    o_ref[...] = _ln(h + t, g[...], b[...])


def _node_update(h, agg, conv):
    spec128 = pl.BlockSpec((_HID, _HID), lambda i: (0, 0))
    row = pl.BlockSpec((1, _HID), lambda i: (0, 0))
    mlp = conv["mlp"]
    ln = conv["post_ln"]
    return pl.pallas_call(
        _node_update_body,
        grid=(_N // _NB,),
        in_specs=[
            pl.BlockSpec((_NB, _HID), lambda i: (i, 0)),
            pl.BlockSpec((_NC, _NB, _HID), lambda i: (0, i, 0)),
            spec128, row, spec128, row, row, row,
        ],
        out_specs=pl.BlockSpec((_NB, _HID), lambda i: (i, 0)),
        out_shape=jax.ShapeDtypeStruct((_N, _HID), jnp.float32),
    )(h, agg, mlp[0]["w"], mlp[0]["b"].reshape(1, -1), mlp[1]["w"],
      mlp[1]["b"].reshape(1, -1), ln["g"].reshape(1, -1), ln["b"].reshape(1, -1))


def _heads_body(h_ref, dw1, db1, dw2, db2, sw1, sb1, sw2, sb2, scale,
                u_ref, ls_ref, s_ref, sf_ref):
    h = h_ref[...]
    t = jnp.maximum(_dot(h, dw1[...]) + db1[...], 0.0)
    raw_u = _dot(t, dw2[...]) + db2[...]
    u_ref[...] = raw_u * scale[0, 0]
    t = jnp.maximum(_dot(h, sw1[...]) + sb1[...], 0.0)
    ls = _dot(t, sw2[...]) + sb2[...]
    ls = jnp.clip(ls, _CLAMP_LO, _CLAMP_HI)
    s = jnp.exp(ls)
    ls_ref[...] = ls
    s_ref[...] = s
    sf_ref[...] = _YIELD_STRESS / (s + 1e-8)


def _heads(h, dp, sp, disp_scale):
    half = _HID // 2
    full = lambda shape: pl.BlockSpec(shape, lambda i: (0, 0))
    col1 = pl.BlockSpec((_NB, 1), lambda i: (i, 0))
    return pl.pallas_call(
        _heads_body,
        grid=(_N // _NB,),
        in_specs=[
            pl.BlockSpec((_NB, _HID), lambda i: (i, 0)),
            full((_HID, half)), full((1, half)), full((half, 3)), full((1, 3)),
            full((_HID, half)), full((1, half)), full((half, 1)), full((1, 1)),
            full((1, 1)),
        ],
        out_specs=[pl.BlockSpec((_NB, 3), lambda i: (i, 0)), col1, col1, col1],
        out_shape=[
            jax.ShapeDtypeStruct((_N, 3), jnp.float32),
            jax.ShapeDtypeStruct((_N, 1), jnp.float32),
            jax.ShapeDtypeStruct((_N, 1), jnp.float32),
            jax.ShapeDtypeStruct((_N, 1), jnp.float32),
        ],
    )(h, dp[0]["w"], dp[0]["b"].reshape(1, -1), dp[1]["w"],
      dp[1]["b"].reshape(1, -1), sp[0]["w"], sp[0]["b"].reshape(1, -1),
      sp[1]["w"], sp[1]["b"].reshape(1, -1), disp_scale.reshape(1, 1))


def _sc_aggregate_body(el_hbm, h_hbm, src_hbm, dst_hbm, zeros_hbm, out_hbm,
                       elbuf, sidx, didx, acc,
                       sem_a0, sem_a1, sem_a2, sem_b0, sem_b1, sem_b2,
                       sem_d0, sem_d1, sem_d2):
    sem_a = (sem_a0, sem_a1, sem_a2)
    sem_b = (sem_b0, sem_b1, sem_b2)
    sem_d = (sem_d0, sem_d1, sem_d2)
    c = lax.axis_index("c")
    s = lax.axis_index("s")
    wid = s * _NC + c
    rbase = pl.multiple_of(16 + s * _ROWS_PER_TILE, 8)

    pltpu.sync_copy(zeros_hbm.at[pl.ds(rbase, _ROWS_PER_TILE)],
                    acc.at[pl.ds(rbase, _ROWS_PER_TILE)])

    @pl.when(s == 0)
    def _zero_head():
        pltpu.sync_copy(zeros_hbm.at[pl.ds(0, 16)], acc.at[pl.ds(0, 16)])

    plsc.subcore_barrier()

    ebase = wid * _EPW
    bbase = wid * _NBLK

    def _a_descs(g, q):
        e0 = ebase + g * _CHUNK
        return (
            pltpu.make_async_copy(el_hbm.at[pl.ds(e0, _CHUNK)], elbuf.at[q],
                                  sem_a[q]),
            pltpu.make_async_copy(src_hbm.at[bbase + g], sidx.at[q],
                                  sem_a[q]),
            pltpu.make_async_copy(dst_hbm.at[bbase + g], didx.at[q],
                                  sem_a[q]),
        )

    def _b_desc(q):
        return pltpu.make_async_copy(h_hbm.at[sidx.at[q, 0]], elbuf.at[q],
                                     sem_b[q])

    def _d_desc(q):
        return pltpu.make_async_copy(elbuf.at[q], acc.at[didx.at[q, 0]],
                                     sem_d[q])

    for d in _a_descs(0, 0):
        d.start()
    for d in _a_descs(1, 1):
        d.start()
    for d in _a_descs(0, 0):
        d.wait()
    _b_desc(0).start(add=True)

    def _triplet(it, carry):
        for j in (0, 1, 2):
            g = it * 3 + j

            @pl.when(g < _NBLK)
            def _body(g=g, j=j):
                _b_desc(j).wait()

                def _rows(rr, cc):
                    for dr in range(4):
                        for k in range(_HID // 16):
                            sl = pl.ds(k * 16, 16)
                            r = rr * 4 + dr
                            elbuf[j, r, sl] = jnp.maximum(elbuf[j, r, sl],
                                                          0.0)
                    return cc

                lax.fori_loop(0, _CHUNK // 4, _rows, 0)
                _d_desc(j).start(add=True)

                @pl.when(g >= 1)
                def _wait_prev_d():
                    _d_desc((j + 2) % 3).wait()

                @pl.when(g + 2 < _NBLK)
                def _next_a():
                    for d in _a_descs(g + 2, (j + 2) % 3):
                        d.start()

                @pl.when(g + 1 < _NBLK)
                def _next_b():
                    for d in _a_descs(g + 1, (j + 1) % 3):
                        d.wait()
                    _b_desc((j + 1) % 3).start(add=True)

        return carry

    lax.fori_loop(0, (_NBLK + 2) // 3, _triplet, 0)
    _d_desc((_NBLK - 1) % 3).wait()
    plsc.subcore_barrier()

    pltpu.sync_copy(acc.at[pl.ds(rbase, _ROWS_PER_TILE)],
                    out_hbm.at[c, pl.ds(rbase, _ROWS_PER_TILE)])

    @pl.when(s == 0)
    def _copy_head():
        pltpu.sync_copy(acc.at[pl.ds(0, 16)], out_hbm.at[c, pl.ds(0, 16)])


def _sc_aggregate(el, h, src3, dst3, zeros):
    mesh = plsc.VectorSubcoreMesh(core_axis_name="c", subcore_axis_name="s",
                                  num_cores=_NC, num_subcores=_NS)
    f = pl.kernel(
        _sc_aggregate_body,
        out_type=jax.ShapeDtypeStruct((_NC, _N, _HID), jnp.float32),
        mesh=mesh,
        scratch_types=[
            pltpu.VMEM((3, _CHUNK, _HID), jnp.float32),
            pltpu.VMEM((3, _BLK_CHUNKS, _CHUNK), jnp.int32),
            pltpu.VMEM((3, _BLK_CHUNKS, _CHUNK), jnp.int32),
            pltpu.VMEM_SHARED((_N, _HID), jnp.float32),
            pltpu.SemaphoreType.DMA,
            pltpu.SemaphoreType.DMA,
            pltpu.SemaphoreType.DMA,
            pltpu.SemaphoreType.DMA,
            pltpu.SemaphoreType.DMA,
            pltpu.SemaphoreType.DMA,
            pltpu.SemaphoreType.DMA,
            pltpu.SemaphoreType.DMA,
            pltpu.SemaphoreType.DMA,
        ],
    )
    return f(el, h, src3, dst3, zeros)


def kernel(x, edge_index, edge_attr, params):
    src3 = edge_index[0].astype(jnp.int32).reshape(_NW * _NBLK, _BLK_CHUNKS,
                                                   _CHUNK)
    dst3 = edge_index[1].astype(jnp.int32).reshape(_NW * _NBLK, _BLK_CHUNKS,
                                                   _CHUNK)

    h = _node_encoder(x, params["node_enc"], params["node_ln"])
    els = _edge_encoder(edge_attr, params["edge_enc"], params["edge_ln"],
                        params["convs"])
    zeros = jnp.zeros((_N, _HID), jnp.float32)

    for li, conv in enumerate(params["convs"]):
        agg = _sc_aggregate(els[li], h, src3, dst3, zeros)
        h = _node_update(h, agg, conv)

    disp_scale = _MIN_DISP_SCALE + jax.nn.softplus(params["log_disp_scale"])
    u, log_s, s_out, safety = _heads(h, params["disp_head"],
                                     params["stress_head"], disp_scale)
    return {
        "displacement": u,
        "stress": s_out,
        "log_stress": log_s,
        "disp_scale": disp_scale,
        "safety_factor": safety,
    }


# speedup vs baseline: 1.0672x; 1.0672x over previous
import functools

import jax
import jax.numpy as jnp
from jax import lax
from jax.experimental import pallas as pl
from jax.experimental.pallas import tpu as pltpu
from jax.experimental.pallas import tpu_sc as plsc

_N = 10000
_E = 320000
_HID = 128
_NUM_LAYERS = 3
_MIN_DISP_SCALE = 0.001
_CLAMP_LO, _CLAMP_HI = 0.0, 30.0
_YIELD_STRESS = 2.5e8

_NC = 2
_NS = 16
_NW = _NC * _NS
_EPW = _E // _NW
_CHUNK = 80
_BLK_CHUNKS = 1
_BLKE = _CHUNK * _BLK_CHUNKS
_NBLK = _EPW // _BLKE
_ROWS_PER_TILE = 624

_NB = 2000
_EB = 2000


def _ln(h, g, b):
    m = jnp.mean(h, axis=-1, keepdims=True)
    v = jnp.mean((h - m) ** 2, axis=-1, keepdims=True)
    return (h - m) / jnp.sqrt(v + 1e-5) * g + b


def _dot(a, b):
    return jnp.dot(a, b, preferred_element_type=jnp.float32)


def _pack_words(x):
    u = jax.lax.bitcast_convert_type(x, jnp.uint32)
    r = (u + 0x7FFF + ((u >> 16) & 1)) >> 16
    lo = jnp.concatenate([r[:, g * 32:g * 32 + 16] for g in range(4)], axis=1)
    hi = jnp.concatenate([r[:, g * 32 + 16:g * 32 + 32] for g in range(4)],
                         axis=1)
    return jax.lax.bitcast_convert_type((hi << 16) | lo, jnp.float32)


def _node_enc_body(x_ref, w1, b1, w2, b2, g, b, o_ref):
    h = jnp.maximum(_dot(x_ref[...], w1[...]) + b1[...], 0.0)
    h = _dot(h, w2[...]) + b2[...]
    o_ref[...] = _ln(h, g[...], b[...])


def _node_encoder(x, p, ln):
    spec128 = pl.BlockSpec((_HID, _HID), lambda i: (0, 0))
    row = pl.BlockSpec((1, _HID), lambda i: (0, 0))
    return pl.pallas_call(
        _node_enc_body,
        grid=(_N // _NB,),
        in_specs=[
            pl.BlockSpec((_NB, x.shape[1]), lambda i: (i, 0)),
            pl.BlockSpec(p[0]["w"].shape, lambda i: (0, 0)),
            row, spec128, row, row, row,
        ],
        out_specs=pl.BlockSpec((_NB, _HID), lambda i: (i, 0)),
        out_shape=jax.ShapeDtypeStruct((_N, _HID), jnp.float32),
    )(x, p[0]["w"], p[0]["b"].reshape(1, -1), p[1]["w"],
      p[1]["b"].reshape(1, -1), ln["g"].reshape(1, -1), ln["b"].reshape(1, -1))


def _edge_enc_body(a_ref, w1, b1, w2, b2, g, b, lw, lb, o1, o2, o3):
    e = jnp.maximum(_dot(a_ref[...], w1[...]) + b1[...], 0.0)
    e = _dot(e, w2[...]) + b2[...]
    e = _ln(e, g[...], b[...])
    o1[...] = _dot(e, lw[0]) + lb[0]
    o2[...] = _dot(e, lw[1]) + lb[1]
    o3[...] = _dot(e, lw[2]) + lb[2]


def _edge_encoder(edge_attr, p, ln, convs):
    lw = jnp.stack([c["edge_lin"]["w"] for c in convs])
    lb = jnp.stack([c["edge_lin"]["b"].reshape(1, -1) for c in convs])
    spec128 = pl.BlockSpec((_HID, _HID), lambda i: (0, 0))
    row = pl.BlockSpec((1, _HID), lambda i: (0, 0))
    out_spec = pl.BlockSpec((_EB, _HID), lambda i: (i, 0))
    out_sh = jax.ShapeDtypeStruct((_E, _HID), jnp.float32)
    return pl.pallas_call(
        _edge_enc_body,
        grid=(_E // _EB,),
        in_specs=[
            pl.BlockSpec((_EB, edge_attr.shape[1]), lambda i: (i, 0)),
            pl.BlockSpec(p[0]["w"].shape, lambda i: (0, 0)),
            row, spec128, row, row, row,
            pl.BlockSpec((_NUM_LAYERS, _HID, _HID), lambda i: (0, 0, 0)),
            pl.BlockSpec((_NUM_LAYERS, 1, _HID), lambda i: (0, 0, 0)),
        ],
        out_specs=[out_spec, out_spec, out_spec],
        out_shape=[out_sh, out_sh, out_sh],
    )(edge_attr, p[0]["w"], p[0]["b"].reshape(1, -1), p[1]["w"],
      p[1]["b"].reshape(1, -1), ln["g"].reshape(1, -1), ln["b"].reshape(1, -1),
      lw, lb)


def _node_update_body(h_ref, agg_ref, w1, b1, w2, b2, g, b, o_ref):
    h = h_ref[...]
    z = h + agg_ref[0] + agg_ref[1]
    t = jnp.maximum(_dot(z, w1[...]) + b1[...], 0.0)
    t = _dot(t, w2[...]) + b2[...]
    t = jnp.maximum(t, 0.0)
    o_ref[...] = _ln(h + t, g[...], b[...])


def _node_update(h, agg, conv):
    spec128 = pl.BlockSpec((_HID, _HID), lambda i: (0, 0))
    row = pl.BlockSpec((1, _HID), lambda i: (0, 0))
    mlp = conv["mlp"]
    ln = conv["post_ln"]
    return pl.pallas_call(
        _node_update_body,
        grid=(_N // _NB,),
        in_specs=[
            pl.BlockSpec((_NB, _HID), lambda i: (i, 0)),
            pl.BlockSpec((_NC, _NB, _HID), lambda i: (0, i, 0)),
            spec128, row, spec128, row, row, row,
        ],
        out_specs=pl.BlockSpec((_NB, _HID), lambda i: (i, 0)),
        out_shape=jax.ShapeDtypeStruct((_N, _HID), jnp.float32),
    )(h, agg, mlp[0]["w"], mlp[0]["b"].reshape(1, -1), mlp[1]["w"],
      mlp[1]["b"].reshape(1, -1), ln["g"].reshape(1, -1), ln["b"].reshape(1, -1))


def _heads_body(h_ref, dw1, db1, dw2, db2, sw1, sb1, sw2, sb2, scale,
                u_ref, ls_ref, s_ref, sf_ref):
    h = h_ref[...]
    t = jnp.maximum(_dot(h, dw1[...]) + db1[...], 0.0)
    raw_u = _dot(t, dw2[...]) + db2[...]
    u_ref[...] = raw_u * scale[0, 0]
    t = jnp.maximum(_dot(h, sw1[...]) + sb1[...], 0.0)
    ls = _dot(t, sw2[...]) + sb2[...]
    ls = jnp.clip(ls, _CLAMP_LO, _CLAMP_HI)
    s = jnp.exp(ls)
    ls_ref[...] = ls
    s_ref[...] = s
    sf_ref[...] = _YIELD_STRESS / (s + 1e-8)


def _heads(h, dp, sp, disp_scale):
    half = _HID // 2
    full = lambda shape: pl.BlockSpec(shape, lambda i: (0, 0))
    col1 = pl.BlockSpec((_NB, 1), lambda i: (i, 0))
    return pl.pallas_call(
        _heads_body,
        grid=(_N // _NB,),
        in_specs=[
            pl.BlockSpec((_NB, _HID), lambda i: (i, 0)),
            full((_HID, half)), full((1, half)), full((half, 3)), full((1, 3)),
            full((_HID, half)), full((1, half)), full((half, 1)), full((1, 1)),
            full((1, 1)),
        ],
        out_specs=[pl.BlockSpec((_NB, 3), lambda i: (i, 0)), col1, col1, col1],
        out_shape=[
            jax.ShapeDtypeStruct((_N, 3), jnp.float32),
            jax.ShapeDtypeStruct((_N, 1), jnp.float32),
            jax.ShapeDtypeStruct((_N, 1), jnp.float32),
            jax.ShapeDtypeStruct((_N, 1), jnp.float32),
        ],
    )(h, dp[0]["w"], dp[0]["b"].reshape(1, -1), dp[1]["w"],
      dp[1]["b"].reshape(1, -1), sp[0]["w"], sp[0]["b"].reshape(1, -1),
      sp[1]["w"], sp[1]["b"].reshape(1, -1), disp_scale.reshape(1, 1))


def _sc_aggregate_body(el_hbm, h_hbm, src_hbm, dst_hbm, zeros_hbm, out_hbm,
                       elbuf, sidx, didx, acc,
                       sem_a0, sem_a1, sem_b0, sem_b1, sem_d0, sem_d1):
    sem_a = (sem_a0, sem_a1)
    sem_b = (sem_b0, sem_b1)
    sem_d = (sem_d0, sem_d1)
    c = lax.axis_index("c")
    s = lax.axis_index("s")
    wid = s * _NC + c
    rbase = pl.multiple_of(16 + s * _ROWS_PER_TILE, 8)

    pltpu.sync_copy(zeros_hbm.at[pl.ds(rbase, _ROWS_PER_TILE)],
                    acc.at[pl.ds(rbase, _ROWS_PER_TILE)])

    @pl.when(s == 0)
    def _zero_head():
        pltpu.sync_copy(zeros_hbm.at[pl.ds(0, 16)], acc.at[pl.ds(0, 16)])

    plsc.subcore_barrier()

    ebase = wid * _EPW
    bbase = wid * _NBLK

    def _a_descs(blk, p):
        e0 = ebase + blk * _BLKE
        return (
            pltpu.make_async_copy(el_hbm.at[pl.ds(e0, _BLKE)], elbuf.at[p],
                                  sem_a[p]),
            pltpu.make_async_copy(src_hbm.at[bbase + blk], sidx.at[p],
                                  sem_a[p]),
            pltpu.make_async_copy(dst_hbm.at[bbase + blk], didx.at[p],
                                  sem_a[p]),
        )

    def _b_descs(p):
        return [pltpu.make_async_copy(
                    h_hbm.at[sidx.at[p, b]],
                    elbuf.at[p, pl.ds(b * _CHUNK, _CHUNK)], sem_b[p])
                for b in range(_BLK_CHUNKS)]

    def _d_descs(p):
        return [pltpu.make_async_copy(
                    elbuf.at[p, pl.ds(b * _CHUNK, _CHUNK)],
                    acc.at[didx.at[p, b]], sem_d[p])
                for b in range(_BLK_CHUNKS)]

    for d in _a_descs(0, 0):
        d.start()

    def _pair(it, carry):
        for p in (0, 1):
            blk = it * 2 + p

            @pl.when(blk < _NBLK)
            def _body(blk=blk, p=p):
                for d in _a_descs(blk, p):
                    d.wait()
                for d in _b_descs(p):
                    d.start(add=True)

                @pl.when(blk >= 1)
                def _wait_prev_d():
                    for d in _d_descs(1 - p):
                        d.wait()

                @pl.when(blk < _NBLK - 1)
                def _next_a():
                    for d in _a_descs(blk + 1, 1 - p):
                        d.start()

                for d in _b_descs(p):
                    d.wait()

                def _row(r, cc):
                    for k in range(_HID // 16):
                        sl = pl.ds(k * 16, 16)
                        elbuf[p, r, sl] = jnp.maximum(elbuf[p, r, sl], 0.0)
                    return cc

                lax.fori_loop(0, _BLKE, _row, 0)
                for d in _d_descs(p):
                    d.start(add=True)

        return carry

    lax.fori_loop(0, (_NBLK + 1) // 2, _pair, 0)
    for d in _d_descs(0):
        d.wait()
    plsc.subcore_barrier()

    pltpu.sync_copy(acc.at[pl.ds(rbase, _ROWS_PER_TILE)],
                    out_hbm.at[c, pl.ds(rbase, _ROWS_PER_TILE)])

    @pl.when(s == 0)
    def _copy_head():
        pltpu.sync_copy(acc.at[pl.ds(0, 16)], out_hbm.at[c, pl.ds(0, 16)])


def _sc_aggregate(el, h, src3, dst3, zeros):
    mesh = plsc.VectorSubcoreMesh(core_axis_name="c", subcore_axis_name="s",
                                  num_cores=_NC, num_subcores=_NS)
    f = pl.kernel(
        _sc_aggregate_body,
        out_type=jax.ShapeDtypeStruct((_NC, _N, _HID), jnp.float32),
        mesh=mesh,
        scratch_types=[
            pltpu.VMEM((2, _BLKE, _HID), jnp.float32),
            pltpu.VMEM((2, _BLK_CHUNKS, _CHUNK), jnp.int32),
            pltpu.VMEM((2, _BLK_CHUNKS, _CHUNK), jnp.int32),
            pltpu.VMEM_SHARED((_N, _HID), jnp.float32),
            pltpu.SemaphoreType.DMA,
            pltpu.SemaphoreType.DMA,
            pltpu.SemaphoreType.DMA,
            pltpu.SemaphoreType.DMA,
            pltpu.SemaphoreType.DMA,
            pltpu.SemaphoreType.DMA,
        ],
    )
    return f(el, h, src3, dst3, zeros)


def kernel(x, edge_index, edge_attr, params):
    src3 = edge_index[0].astype(jnp.int32).reshape(_NW * _NBLK, _BLK_CHUNKS,
                                                   _CHUNK)
    dst3 = edge_index[1].astype(jnp.int32).reshape(_NW * _NBLK, _BLK_CHUNKS,
                                                   _CHUNK)

    h = _node_encoder(x, params["node_enc"], params["node_ln"])
    els = _edge_encoder(edge_attr, params["edge_enc"], params["edge_ln"],
                        params["convs"])
    zeros = jnp.zeros((_N, _HID), jnp.float32)

    for li, conv in enumerate(params["convs"]):
        agg = _sc_aggregate(els[li], h, src3, dst3, zeros)
        h = _node_update(h, agg, conv)

    disp_scale = _MIN_DISP_SCALE + jax.nn.softplus(params["log_disp_scale"])
    u, log_s, s_out, safety = _heads(h, params["disp_head"],
                                     params["stress_head"], disp_scale)
    return {
        "displacement": u,
        "stress": s_out,
        "log_stress": log_s,
        "disp_scale": disp_scale,
        "safety_factor": safety,
    }


# speedup vs baseline: 1.1454x; 1.0733x over previous
import functools

import jax
import jax.numpy as jnp
from jax import lax
from jax.experimental import pallas as pl
from jax.experimental.pallas import tpu as pltpu
from jax.experimental.pallas import tpu_sc as plsc

_N = 10000
_E = 320000
_HID = 128
_NUM_LAYERS = 3
_MIN_DISP_SCALE = 0.001
_CLAMP_LO, _CLAMP_HI = 0.0, 30.0
_YIELD_STRESS = 2.5e8

_NC = 2
_NS = 16
_NW = _NC * _NS
_CHUNK = 128
_BLK_CHUNKS = 1
_BLKE = _CHUNK * _BLK_CHUNKS
_NCH_TOT = _E // _CHUNK
_CPW = _NCH_TOT // _NW
_EXTRA = _NCH_TOT - _NW * _CPW
_ROWS_PER_TILE = 624

_NB = 2000
_EB = 2000


def _ln(h, g, b):
    m = jnp.mean(h, axis=-1, keepdims=True)
    v = jnp.mean((h - m) ** 2, axis=-1, keepdims=True)
    return (h - m) / jnp.sqrt(v + 1e-5) * g + b


def _dot(a, b):
    return jnp.dot(a, b, preferred_element_type=jnp.float32)


def _pack_words(x):
    u = jax.lax.bitcast_convert_type(x, jnp.uint32)
    r = (u + 0x7FFF + ((u >> 16) & 1)) >> 16
    lo = jnp.concatenate([r[:, g * 32:g * 32 + 16] for g in range(4)], axis=1)
    hi = jnp.concatenate([r[:, g * 32 + 16:g * 32 + 32] for g in range(4)],
                         axis=1)
    return jax.lax.bitcast_convert_type((hi << 16) | lo, jnp.float32)


def _node_enc_body(x_ref, w1, b1, w2, b2, g, b, o_ref):
    h = jnp.maximum(_dot(x_ref[...], w1[...]) + b1[...], 0.0)
    h = _dot(h, w2[...]) + b2[...]
    o_ref[...] = _ln(h, g[...], b[...])


def _node_encoder(x, p, ln):
    spec128 = pl.BlockSpec((_HID, _HID), lambda i: (0, 0))
    row = pl.BlockSpec((1, _HID), lambda i: (0, 0))
    return pl.pallas_call(
        _node_enc_body,
        grid=(_N // _NB,),
        in_specs=[
            pl.BlockSpec((_NB, x.shape[1]), lambda i: (i, 0)),
            pl.BlockSpec(p[0]["w"].shape, lambda i: (0, 0)),
            row, spec128, row, row, row,
        ],
        out_specs=pl.BlockSpec((_NB, _HID), lambda i: (i, 0)),
        out_shape=jax.ShapeDtypeStruct((_N, _HID), jnp.float32),
    )(x, p[0]["w"], p[0]["b"].reshape(1, -1), p[1]["w"],
      p[1]["b"].reshape(1, -1), ln["g"].reshape(1, -1), ln["b"].reshape(1, -1))


def _edge_enc_body(a_ref, w1, b1, w2, b2, g, b, lw, lb, o1, o2, o3):
    e = jnp.maximum(_dot(a_ref[...], w1[...]) + b1[...], 0.0)
    e = _dot(e, w2[...]) + b2[...]
    e = _ln(e, g[...], b[...])
    o1[...] = _dot(e, lw[0]) + lb[0]
    o2[...] = _dot(e, lw[1]) + lb[1]
    o3[...] = _dot(e, lw[2]) + lb[2]


def _edge_encoder(edge_attr, p, ln, convs):
    lw = jnp.stack([c["edge_lin"]["w"] for c in convs])
    lb = jnp.stack([c["edge_lin"]["b"].reshape(1, -1) for c in convs])
    spec128 = pl.BlockSpec((_HID, _HID), lambda i: (0, 0))
    row = pl.BlockSpec((1, _HID), lambda i: (0, 0))
    out_spec = pl.BlockSpec((_EB, _HID), lambda i: (i, 0))
    out_sh = jax.ShapeDtypeStruct((_E, _HID), jnp.float32)
    return pl.pallas_call(
        _edge_enc_body,
        grid=(_E // _EB,),
        in_specs=[
            pl.BlockSpec((_EB, edge_attr.shape[1]), lambda i: (i, 0)),
            pl.BlockSpec(p[0]["w"].shape, lambda i: (0, 0)),
            row, spec128, row, row, row,
            pl.BlockSpec((_NUM_LAYERS, _HID, _HID), lambda i: (0, 0, 0)),
            pl.BlockSpec((_NUM_LAYERS, 1, _HID), lambda i: (0, 0, 0)),
        ],
        out_specs=[out_spec, out_spec, out_spec],
        out_shape=[out_sh, out_sh, out_sh],
    )(edge_attr, p[0]["w"], p[0]["b"].reshape(1, -1), p[1]["w"],
      p[1]["b"].reshape(1, -1), ln["g"].reshape(1, -1), ln["b"].reshape(1, -1),
      lw, lb)


def _node_update_body(h_ref, agg_ref, w1, b1, w2, b2, g, b, o_ref):
    h = h_ref[...]
    z = h + agg_ref[0] + agg_ref[1]
    t = jnp.maximum(_dot(z, w1[...]) + b1[...], 0.0)
    t = _dot(t, w2[...]) + b2[...]
    t = jnp.maximum(t, 0.0)
    o_ref[...] = _ln(h + t, g[...], b[...])


def _node_update(h, agg, conv):
    spec128 = pl.BlockSpec((_HID, _HID), lambda i: (0, 0))
    row = pl.BlockSpec((1, _HID), lambda i: (0, 0))
    mlp = conv["mlp"]
    ln = conv["post_ln"]
    return pl.pallas_call(
        _node_update_body,
        grid=(_N // _NB,),
        in_specs=[
            pl.BlockSpec((_NB, _HID), lambda i: (i, 0)),
            pl.BlockSpec((_NC, _NB, _HID), lambda i: (0, i, 0)),
            spec128, row, spec128, row, row, row,
        ],
        out_specs=pl.BlockSpec((_NB, _HID), lambda i: (i, 0)),
        out_shape=jax.ShapeDtypeStruct((_N, _HID), jnp.float32),
    )(h, agg, mlp[0]["w"], mlp[0]["b"].reshape(1, -1), mlp[1]["w"],
      mlp[1]["b"].reshape(1, -1), ln["g"].reshape(1, -1), ln["b"].reshape(1, -1))


def _heads_body(h_ref, dw1, db1, dw2, db2, sw1, sb1, sw2, sb2, scale,
                u_ref, ls_ref, s_ref, sf_ref):
    h = h_ref[...]
    t = jnp.maximum(_dot(h, dw1[...]) + db1[...], 0.0)
    raw_u = _dot(t, dw2[...]) + db2[...]
    u_ref[...] = raw_u * scale[0, 0]
    t = jnp.maximum(_dot(h, sw1[...]) + sb1[...], 0.0)
    ls = _dot(t, sw2[...]) + sb2[...]
    ls = jnp.clip(ls, _CLAMP_LO, _CLAMP_HI)
    s = jnp.exp(ls)
    ls_ref[...] = ls
    s_ref[...] = s
    sf_ref[...] = _YIELD_STRESS / (s + 1e-8)


def _heads(h, dp, sp, disp_scale):
    half = _HID // 2
    full = lambda shape: pl.BlockSpec(shape, lambda i: (0, 0))
    col1 = pl.BlockSpec((_NB, 1), lambda i: (i, 0))
    return pl.pallas_call(
        _heads_body,
        grid=(_N // _NB,),
        in_specs=[
            pl.BlockSpec((_NB, _HID), lambda i: (i, 0)),
            full((_HID, half)), full((1, half)), full((half, 3)), full((1, 3)),
            full((_HID, half)), full((1, half)), full((half, 1)), full((1, 1)),
            full((1, 1)),
        ],
        out_specs=[pl.BlockSpec((_NB, 3), lambda i: (i, 0)), col1, col1, col1],
        out_shape=[
            jax.ShapeDtypeStruct((_N, 3), jnp.float32),
            jax.ShapeDtypeStruct((_N, 1), jnp.float32),
            jax.ShapeDtypeStruct((_N, 1), jnp.float32),
            jax.ShapeDtypeStruct((_N, 1), jnp.float32),
        ],
    )(h, dp[0]["w"], dp[0]["b"].reshape(1, -1), dp[1]["w"],
      dp[1]["b"].reshape(1, -1), sp[0]["w"], sp[0]["b"].reshape(1, -1),
      sp[1]["w"], sp[1]["b"].reshape(1, -1), disp_scale.reshape(1, 1))


def _sc_aggregate_body(el_hbm, h_hbm, src_hbm, dst_hbm, zeros_hbm, out_hbm,
                       elbuf, sidx, didx, acc,
                       sem_a0, sem_a1, sem_b0, sem_b1, sem_d0, sem_d1):
    sem_a = (sem_a0, sem_a1)
    sem_b = (sem_b0, sem_b1)
    sem_d = (sem_d0, sem_d1)
    c = lax.axis_index("c")
    s = lax.axis_index("s")
    wid = s * _NC + c
    cstart = _CPW * wid + jnp.minimum(wid, _EXTRA)
    cnt = _CPW + (wid < _EXTRA).astype(jnp.int32)
    rbase = pl.multiple_of(16 + s * _ROWS_PER_TILE, 8)

    pltpu.sync_copy(zeros_hbm.at[pl.ds(rbase, _ROWS_PER_TILE)],
                    acc.at[pl.ds(rbase, _ROWS_PER_TILE)])

    @pl.when(s == 0)
    def _zero_head():
        pltpu.sync_copy(zeros_hbm.at[pl.ds(0, 16)], acc.at[pl.ds(0, 16)])

    plsc.subcore_barrier()

    def _a_descs(blk, p):
        e0 = (cstart + blk) * _BLKE
        return (
            pltpu.make_async_copy(el_hbm.at[pl.ds(e0, _BLKE)], elbuf.at[p],
                                  sem_a[p]),
            pltpu.make_async_copy(src_hbm.at[cstart + blk], sidx.at[p],
                                  sem_a[p]),
            pltpu.make_async_copy(dst_hbm.at[cstart + blk], didx.at[p],
                                  sem_a[p]),
        )

    def _b_descs(p):
        return [pltpu.make_async_copy(
                    h_hbm.at[sidx.at[p, b]],
                    elbuf.at[p, pl.ds(b * _CHUNK, _CHUNK)], sem_b[p])
                for b in range(_BLK_CHUNKS)]

    def _d_descs(p):
        return [pltpu.make_async_copy(
                    elbuf.at[p, pl.ds(b * _CHUNK, _CHUNK)],
                    acc.at[didx.at[p, b]], sem_d[p])
                for b in range(_BLK_CHUNKS)]

    for d in _a_descs(0, 0):
        d.start()

    def _pair(it, carry):
        for p in (0, 1):
            blk = it * 2 + p

            @pl.when(blk < cnt)
            def _body(blk=blk, p=p):
                for d in _a_descs(blk, p):
                    d.wait()
                for d in _b_descs(p):
                    d.start(add=True)

                @pl.when(blk >= 1)
                def _wait_prev_d():
                    for d in _d_descs(1 - p):
                        d.wait()

                @pl.when(blk < cnt - 1)
                def _next_a():
                    for d in _a_descs(blk + 1, 1 - p):
                        d.start()

                for d in _b_descs(p):
                    d.wait()

                def _row(r, cc):
                    for k in range(_HID // 16):
                        sl = pl.ds(k * 16, 16)
                        elbuf[p, r, sl] = jnp.maximum(elbuf[p, r, sl], 0.0)
                    return cc

                lax.fori_loop(0, _BLKE, _row, 0)
                for d in _d_descs(p):
                    d.start(add=True)

        return carry

    lax.fori_loop(0, (_CPW + 2) // 2, _pair, 0)

    @pl.when(cnt % 2 == 1)
    def _drain_even():
        for d in _d_descs(0):
            d.wait()

    @pl.when(cnt % 2 == 0)
    def _drain_odd():
        for d in _d_descs(1):
            d.wait()

    plsc.subcore_barrier()

    pltpu.sync_copy(acc.at[pl.ds(rbase, _ROWS_PER_TILE)],
                    out_hbm.at[c, pl.ds(rbase, _ROWS_PER_TILE)])

    @pl.when(s == 0)
    def _copy_head():
        pltpu.sync_copy(acc.at[pl.ds(0, 16)], out_hbm.at[c, pl.ds(0, 16)])


def _sc_aggregate(el, h, src3, dst3, zeros):
    mesh = plsc.VectorSubcoreMesh(core_axis_name="c", subcore_axis_name="s",
                                  num_cores=_NC, num_subcores=_NS)
    f = pl.kernel(
        _sc_aggregate_body,
        out_type=jax.ShapeDtypeStruct((_NC, _N, _HID), jnp.float32),
        mesh=mesh,
        scratch_types=[
            pltpu.VMEM((2, _BLKE, _HID), jnp.float32),
            pltpu.VMEM((2, _BLK_CHUNKS, _CHUNK), jnp.int32),
            pltpu.VMEM((2, _BLK_CHUNKS, _CHUNK), jnp.int32),
            pltpu.VMEM_SHARED((_N, _HID), jnp.float32),
            pltpu.SemaphoreType.DMA,
            pltpu.SemaphoreType.DMA,
            pltpu.SemaphoreType.DMA,
            pltpu.SemaphoreType.DMA,
            pltpu.SemaphoreType.DMA,
            pltpu.SemaphoreType.DMA,
        ],
    )
    return f(el, h, src3, dst3, zeros)


def kernel(x, edge_index, edge_attr, params):
    src3 = edge_index[0].astype(jnp.int32).reshape(_NCH_TOT, _BLK_CHUNKS,
                                                   _CHUNK)
    dst3 = edge_index[1].astype(jnp.int32).reshape(_NCH_TOT, _BLK_CHUNKS,
                                                   _CHUNK)

    h = _node_encoder(x, params["node_enc"], params["node_ln"])
    els = _edge_encoder(edge_attr, params["edge_enc"], params["edge_ln"],
                        params["convs"])
    zeros = jnp.zeros((_N, _HID), jnp.float32)

    for li, conv in enumerate(params["convs"]):
        agg = _sc_aggregate(els[li], h, src3, dst3, zeros)
        h = _node_update(h, agg, conv)

    disp_scale = _MIN_DISP_SCALE + jax.nn.softplus(params["log_disp_scale"])
    u, log_s, s_out, safety = _heads(h, params["disp_head"],
                                     params["stress_head"], disp_scale)
    return {
        "displacement": u,
        "stress": s_out,
        "log_stress": log_s,
        "disp_scale": disp_scale,
        "safety_factor": safety,
    }
